# baseline (speedup 1.0000x reference)
import functools

import jax
import jax.numpy as jnp
from jax.experimental import pallas as pl

NV = 51
NP = 56


def _leaky(h):
    return jnp.where(h >= 0, h, 0.01 * h)


def _edge_head(x, ws, bs, scales, shifts, depth):
    a = jnp.abs(x[:, None, :] - x[None, :, :]).reshape(NP * NP, x.shape[1])
    h = a
    for l in range(depth):
        h = jnp.dot(h.astype(jnp.bfloat16), ws[l][...],
                    preferred_element_type=jnp.float32) + bs[l][0]
        if l < depth - 1:
            h = _leaky(h * scales[l][0] + shifts[l][0])
    return h


def _stats_kernel(*args, depth):
    nw = depth
    x_ref = args[0]
    ws = args[1:1 + nw]
    bs = args[1 + nw:1 + 2 * nw]
    sc = args[1 + 2 * nw:1 + 2 * nw + (depth - 1)]
    sh = args[1 + 2 * nw + (depth - 1):1 + 2 * nw + 2 * (depth - 1)]
    sum_ref, sq_ref = args[-2], args[-1]
    h = _edge_head(x_ref[0], ws, bs, sc, sh, depth)
    r = jax.lax.broadcasted_iota(jnp.int32, (NP * NP, 1), 0)
    n = r // NP
    m = r - n * NP
    mask = ((n < NV) & (m < NV)).astype(jnp.float32)
    hm = h * mask
    sum_ref[0, 0, :] = jnp.sum(hm, axis=0)
    sq_ref[0, 0, :] = jnp.sum(hm * h, axis=0)


def _final_kernel(*args):
    x_ref = args[0]
    ws = args[1:6]
    bs = args[6:11]
    sc = args[11:15]
    sh = args[15:19]
    wg1, wg2, bg = args[19], args[20], args[21]
    gout_ref, gsum_ref, gsq_ref = args[-3], args[-2], args[-1]
    x = x_ref[0]
    h5 = _edge_head(x, ws, bs, sc, sh, 5).reshape(NP, NP)
    ii = jax.lax.broadcasted_iota(jnp.int32, (NP, NP), 0)
    jj = jax.lax.broadcasted_iota(jnp.int32, (NP, NP), 1)
    le = h5 + jnp.where(ii == jj, -1e8, 0.0) + jnp.where(jj >= NV, -1e9, 0.0)
    mx = jnp.max(le, axis=1, keepdims=True)
    e = jnp.exp(le - mx)
    w = e / jnp.sum(e, axis=1, keepdims=True)
    y = jnp.dot(w, x, preferred_element_type=jnp.float32)
    gout = (jnp.dot(x, wg1[...], preferred_element_type=jnp.float32)
            + jnp.dot(y, wg2[...], preferred_element_type=jnp.float32) + bg[0])
    ni = jax.lax.broadcasted_iota(jnp.int32, (NP, 1), 0)
    vmask = (ni < NV).astype(jnp.float32)
    gout = gout * vmask
    gout_ref[0] = gout
    gsum_ref[0, 0, :] = jnp.sum(gout, axis=0)
    gsq_ref[0, 0, :] = jnp.sum(gout * gout, axis=0)


def _final0_kernel(*args):
    x_ref = args[0]
    ws = args[1:6]
    bs = args[6:11]
    sc = args[11:15]
    sh = args[15:19]
    wg1, wg2, bg = args[19], args[20], args[21]
    logit_ref, sig_ref = args[-2], args[-1]
    x = x_ref[0]
    a = jnp.abs(x[0:1, :] - x)
    h = a
    for l in range(5):
        h = jnp.dot(h.astype(jnp.bfloat16), ws[l][...],
                    preferred_element_type=jnp.float32) + bs[l][0]
        if l < 4:
            h = _leaky(h * sc[l][0] + sh[l][0])
    ri = jax.lax.broadcasted_iota(jnp.int32, (NP, 1), 0)
    le = h + jnp.where(ri == 0, -1e8, 0.0) + jnp.where(ri >= NV, -1e9, 0.0)
    mx = jnp.max(le, axis=0, keepdims=True)
    e = jnp.exp(le - mx)
    w = e / jnp.sum(e, axis=0, keepdims=True)
    y = jax.lax.dot_general(w, x, (((0,), (0,)), ((), ())),
                            preferred_element_type=jnp.float32)
    gl = (jnp.dot(x[0:1, :], wg1[...], preferred_element_type=jnp.float32)
          + jnp.dot(y, wg2[...], preferred_element_type=jnp.float32) + bg[0])
    logit_ref[0, 0, :] = gl[0]
    sig_ref[0, 0, :] = (1.0 / (1.0 + jnp.exp(-gl)))[0]


def _bn_act_kernel(g_ref, scale_ref, shift_ref, out_ref):
    ni = jax.lax.broadcasted_iota(jnp.int32, (NP, 1), 0)
    vmask = (ni < NV).astype(jnp.float32)
    out_ref[0] = _leaky(g_ref[0] * scale_ref[0] + shift_ref[0]) * vmask


def _full_spec(shape):
    nd = len(shape)
    return pl.BlockSpec(shape, lambda b: (0,) * nd)


def _run_stats(xp, ws, bs, scales, shifts, depth):
    B, _, F = xp.shape
    C = ws[depth - 1].shape[1]
    ins = [xp] + list(ws[:depth]) + list(bs[:depth]) + \
        list(scales[:depth - 1]) + list(shifts[:depth - 1])
    in_specs = [pl.BlockSpec((1, NP, F), lambda b: (b, 0, 0))] + \
        [_full_spec(a.shape) for a in ins[1:]]
    out_shape = [jax.ShapeDtypeStruct((B, 1, C), jnp.float32)] * 2
    out_specs = [pl.BlockSpec((1, 1, C), lambda b: (b, 0, 0))] * 2
    s, sq = pl.pallas_call(
        functools.partial(_stats_kernel, depth=depth),
        grid=(B,), in_specs=in_specs, out_specs=out_specs,
        out_shape=out_shape)(*ins)
    return jnp.sum(s[:, 0, :], axis=0), jnp.sum(sq[:, 0, :], axis=0)


def _finalize(s, sq, cnt, g, beta):
    mean = s / cnt
    var = sq / cnt - mean * mean
    scale = (g * jax.lax.rsqrt(var + 1e-5)).reshape(1, -1)
    shift = (beta - mean * scale[0]).reshape(1, -1)
    return scale, shift


def _wcompute_gconv(xp, wc, gcp, last):
    B, _, F = xp.shape
    ws = [w.astype(jnp.bfloat16) for w in wc["w"]]
    bs = [b.reshape(1, -1) for b in wc["b"]]
    cnt = float(B * NV * NV)
    scales, shifts = [], []
    for d in range(1, 5):
        s, sq = _run_stats(xp, ws, bs, scales, shifts, d)
        sc, sh = _finalize(s, sq, cnt, wc["g"][d - 1], wc["beta"][d - 1])
        scales.append(sc)
        shifts.append(sh)
    wg1, wg2 = gcp["w"][:F], gcp["w"][F:]
    bg = gcp["b"].reshape(1, -1)
    Fo = wg1.shape[1]
    ins = [xp] + list(ws) + bs + scales + shifts + [wg1, wg2, bg]
    in_specs = [pl.BlockSpec((1, NP, F), lambda b: (b, 0, 0))] + \
        [_full_spec(a.shape) for a in ins[1:]]
    if last:
        out_shape = [jax.ShapeDtypeStruct((B, 1, Fo), jnp.float32)] * 2
        out_specs = [pl.BlockSpec((1, 1, Fo), lambda b: (b, 0, 0))] * 2
        logits, sig = pl.pallas_call(
            _final0_kernel, grid=(B,), in_specs=in_specs,
            out_specs=out_specs, out_shape=out_shape)(*ins)
        return logits[:, 0, :], sig[:, 0, :]
    out_shape = [jax.ShapeDtypeStruct((B, NP, Fo), jnp.float32),
                 jax.ShapeDtypeStruct((B, 1, Fo), jnp.float32),
                 jax.ShapeDtypeStruct((B, 1, Fo), jnp.float32)]
    out_specs = [pl.BlockSpec((1, NP, Fo), lambda b: (b, 0, 0)),
                 pl.BlockSpec((1, 1, Fo), lambda b: (b, 0, 0)),
                 pl.BlockSpec((1, 1, Fo), lambda b: (b, 0, 0))]
    gout, gs, gq = pl.pallas_call(
        _final_kernel, grid=(B,), in_specs=in_specs,
        out_specs=out_specs, out_shape=out_shape)(*ins)
    gs = jnp.sum(gs[:, 0, :], axis=0)
    gq = jnp.sum(gq[:, 0, :], axis=0)
    gscale, gshift = _finalize(gs, gq, float(B * NV), gcp["g"], gcp["beta"])
    act = pl.pallas_call(
        _bn_act_kernel, grid=(B,),
        in_specs=[pl.BlockSpec((1, NP, Fo), lambda b: (b, 0, 0)),
                  _full_spec(gscale.shape), _full_spec(gshift.shape)],
        out_specs=pl.BlockSpec((1, NP, Fo), lambda b: (b, 0, 0)),
        out_shape=jax.ShapeDtypeStruct((B, NP, Fo), jnp.float32))(
            gout, gscale, gshift)
    return act


def kernel(z, zi_s, labels_yi, params):
    B = z.shape[0]
    zero_pad = jnp.zeros((1, B, labels_yi.shape[2]), dtype=labels_yi.dtype)
    lab_all = jnp.concatenate([zero_pad, labels_yi], axis=0)
    z_all = jnp.concatenate([z[None], zi_s], axis=0)
    nodes = jnp.transpose(jnp.concatenate([z_all, lab_all], axis=2), (1, 0, 2))
    xp = jnp.pad(nodes, ((0, 0), (0, NP - NV), (0, 0)))
    for i in range(2):
        act = _wcompute_gconv(xp, params["wc"][i], params["gc"][i], last=False)
        xp = jnp.concatenate([xp, act], axis=2)
    logits, sig = _wcompute_gconv(xp, params["wc"][2], params["gc"][2],
                                  last=True)
    return (sig, logits)


# speedup vs baseline: 1.2472x; 1.2472x over previous
import functools

import jax
import jax.numpy as jnp
from jax.experimental import pallas as pl

NV = 51
NP = 56


def _leaky(h):
    return jnp.maximum(h, 0.01 * h)


def _edge_head(x, ws, bs, depth):
    a = jnp.abs(x[:, None, :] - x[None, :, :]).reshape(NP * NP, x.shape[1])
    h = a
    for l in range(depth):
        h = jnp.dot(h, ws[l][...], preferred_element_type=jnp.float32) + bs[l][0]
        if l < depth - 1:
            h = _leaky(h)
    return h


def _pair_mask_row():
    r = jax.lax.broadcasted_iota(jnp.int32, (1, NP * NP), 1)
    n = r // NP
    m = r - n * NP
    return ((n < NV) & (m < NV)).astype(jnp.float32)


def _stats_kernel(*args, depth):
    x_ref = args[0]
    ws = args[1:1 + depth]
    bs = args[1 + depth:1 + 2 * depth]
    sum_ref, sq_ref = args[-2], args[-1]
    h = _edge_head(x_ref[0], ws, bs, depth)
    mask = _pair_mask_row()
    sum_ref[0, 0, :] = jnp.dot(mask, h, preferred_element_type=jnp.float32)[0]
    sq_ref[0, 0, :] = jnp.dot(mask, h * h,
                              preferred_element_type=jnp.float32)[0]


def _final_kernel(*args):
    x_ref = args[0]
    ws = args[1:6]
    bs = args[6:11]
    wg1, wg2, bg = args[11], args[12], args[13]
    gout_ref, gsum_ref, gsq_ref = args[-3], args[-2], args[-1]
    x = x_ref[0]
    h5 = _edge_head(x, ws, bs, 5).reshape(NP, NP)
    ii = jax.lax.broadcasted_iota(jnp.int32, (NP, NP), 0)
    jj = jax.lax.broadcasted_iota(jnp.int32, (NP, NP), 1)
    le = h5 + jnp.where(ii == jj, -1e8, 0.0) + jnp.where(jj >= NV, -1e9, 0.0)
    mx = jnp.max(le, axis=1, keepdims=True)
    e = jnp.exp(le - mx)
    w = e / jnp.sum(e, axis=1, keepdims=True)
    y = jnp.dot(w, x, preferred_element_type=jnp.float32)
    gout = (jnp.dot(x, wg1[...], preferred_element_type=jnp.float32)
            + jnp.dot(y, wg2[...], preferred_element_type=jnp.float32) + bg[0])
    ni = jax.lax.broadcasted_iota(jnp.int32, (NP, 1), 0)
    vmask = (ni < NV).astype(jnp.float32)
    gout = gout * vmask
    gout_ref[0] = gout
    gsum_ref[0, 0, :] = jnp.sum(gout, axis=0)
    gsq_ref[0, 0, :] = jnp.sum(gout * gout, axis=0)


def _final0_kernel(*args):
    x_ref = args[0]
    ws = args[1:6]
    bs = args[6:11]
    wg1, wg2, bg = args[11], args[12], args[13]
    logit_ref, sig_ref = args[-2], args[-1]
    x = x_ref[0]
    a = jnp.abs(x[0:1, :] - x)
    h = a
    for l in range(5):
        h = jnp.dot(h, ws[l][...], preferred_element_type=jnp.float32) + bs[l][0]
        if l < 4:
            h = _leaky(h)
    ri = jax.lax.broadcasted_iota(jnp.int32, (NP, 1), 0)
    le = h + jnp.where(ri == 0, -1e8, 0.0) + jnp.where(ri >= NV, -1e9, 0.0)
    mx = jnp.max(le, axis=0, keepdims=True)
    e = jnp.exp(le - mx)
    w = e / jnp.sum(e, axis=0, keepdims=True)
    y = jax.lax.dot_general(w, x, (((0,), (0,)), ((), ())),
                            preferred_element_type=jnp.float32)
    gl = (jnp.dot(x[0:1, :], wg1[...], preferred_element_type=jnp.float32)
          + jnp.dot(y, wg2[...], preferred_element_type=jnp.float32) + bg[0])
    logit_ref[0, 0, :] = gl[0]
    sig_ref[0, 0, :] = (1.0 / (1.0 + jnp.exp(-gl)))[0]


def _bn_act_kernel(g_ref, scale_ref, shift_ref, out_ref):
    ni = jax.lax.broadcasted_iota(jnp.int32, (NP, 1), 0)
    vmask = (ni < NV).astype(jnp.float32)
    out_ref[0] = _leaky(g_ref[0] * scale_ref[0] + shift_ref[0]) * vmask


def _full_spec(shape):
    nd = len(shape)
    return pl.BlockSpec(shape, lambda b: (0,) * nd)


def _fold(ws, bs, scales, shifts, depth):
    wf, bf = [], []
    for l in range(depth - 1):
        wf.append(ws[l] * scales[l])
        bf.append(bs[l] * scales[l] + shifts[l])
    wf.append(ws[depth - 1])
    bf.append(bs[depth - 1])
    return wf, bf


def _run_stats(xp, ws, bs, scales, shifts, depth):
    B, _, F = xp.shape
    C = ws[depth - 1].shape[1]
    wf, bf = _fold(ws, bs, scales, shifts, depth)
    ins = [xp] + wf + bf
    in_specs = [pl.BlockSpec((1, NP, F), lambda b: (b, 0, 0))] + \
        [_full_spec(a.shape) for a in ins[1:]]
    out_shape = [jax.ShapeDtypeStruct((B, 1, C), jnp.float32)] * 2
    out_specs = [pl.BlockSpec((1, 1, C), lambda b: (b, 0, 0))] * 2
    s, sq = pl.pallas_call(
        functools.partial(_stats_kernel, depth=depth),
        grid=(B,), in_specs=in_specs, out_specs=out_specs,
        out_shape=out_shape)(*ins)
    return jnp.sum(s[:, 0, :], axis=0), jnp.sum(sq[:, 0, :], axis=0)


def _finalize(s, sq, cnt, g, beta):
    mean = s / cnt
    var = sq / cnt - mean * mean
    scale = (g * jax.lax.rsqrt(var + 1e-5)).reshape(1, -1)
    shift = (beta - mean * scale[0]).reshape(1, -1)
    return scale, shift


def _wcompute_gconv(xp, wc, gcp, last):
    B, _, F = xp.shape
    ws = wc["w"]
    bs = [b.reshape(1, -1) for b in wc["b"]]
    cnt = float(B * NV * NV)
    scales, shifts = [], []
    for d in range(1, 5):
        s, sq = _run_stats(xp, ws, bs, scales, shifts, d)
        sc, sh = _finalize(s, sq, cnt, wc["g"][d - 1], wc["beta"][d - 1])
        scales.append(sc)
        shifts.append(sh)
    wg1, wg2 = gcp["w"][:F], gcp["w"][F:]
    bg = gcp["b"].reshape(1, -1)
    Fo = wg1.shape[1]
    wf, bf = _fold(ws, bs, scales, shifts, 5)
    ins = [xp] + wf + bf + [wg1, wg2, bg]
    in_specs = [pl.BlockSpec((1, NP, F), lambda b: (b, 0, 0))] + \
        [_full_spec(a.shape) for a in ins[1:]]
    if last:
        out_shape = [jax.ShapeDtypeStruct((B, 1, Fo), jnp.float32)] * 2
        out_specs = [pl.BlockSpec((1, 1, Fo), lambda b: (b, 0, 0))] * 2
        logits, sig = pl.pallas_call(
            _final0_kernel, grid=(B,), in_specs=in_specs,
            out_specs=out_specs, out_shape=out_shape)(*ins)
        return logits[:, 0, :], sig[:, 0, :]
    out_shape = [jax.ShapeDtypeStruct((B, NP, Fo), jnp.float32),
                 jax.ShapeDtypeStruct((B, 1, Fo), jnp.float32),
                 jax.ShapeDtypeStruct((B, 1, Fo), jnp.float32)]
    out_specs = [pl.BlockSpec((1, NP, Fo), lambda b: (b, 0, 0)),
                 pl.BlockSpec((1, 1, Fo), lambda b: (b, 0, 0)),
                 pl.BlockSpec((1, 1, Fo), lambda b: (b, 0, 0))]
    gout, gs, gq = pl.pallas_call(
        _final_kernel, grid=(B,), in_specs=in_specs,
        out_specs=out_specs, out_shape=out_shape)(*ins)
    gs = jnp.sum(gs[:, 0, :], axis=0)
    gq = jnp.sum(gq[:, 0, :], axis=0)
    gscale, gshift = _finalize(gs, gq, float(B * NV), gcp["g"], gcp["beta"])
    act = pl.pallas_call(
        _bn_act_kernel, grid=(B,),
        in_specs=[pl.BlockSpec((1, NP, Fo), lambda b: (b, 0, 0)),
                  _full_spec(gscale.shape), _full_spec(gshift.shape)],
        out_specs=pl.BlockSpec((1, NP, Fo), lambda b: (b, 0, 0)),
        out_shape=jax.ShapeDtypeStruct((B, NP, Fo), jnp.float32))(
            gout, gscale, gshift)
    return act


def kernel(z, zi_s, labels_yi, params):
    B = z.shape[0]
    zero_pad = jnp.zeros((1, B, labels_yi.shape[2]), dtype=labels_yi.dtype)
    lab_all = jnp.concatenate([zero_pad, labels_yi], axis=0)
    z_all = jnp.concatenate([z[None], zi_s], axis=0)
    nodes = jnp.transpose(jnp.concatenate([z_all, lab_all], axis=2), (1, 0, 2))
    xp = jnp.pad(nodes, ((0, 0), (0, NP - NV), (0, 0)))
    for i in range(2):
        act = _wcompute_gconv(xp, params["wc"][i], params["gc"][i], last=False)
        xp = jnp.concatenate([xp, act], axis=2)
    logits, sig = _wcompute_gconv(xp, params["wc"][2], params["gc"][2],
                                  last=True)
    return (sig, logits)


# speedup vs baseline: 1.4865x; 1.1919x over previous
import jax
import jax.numpy as jnp
from jax.experimental import pallas as pl

NV = 51
NP = 56
NR = NP * NP
F32 = jnp.float32
BF16 = jnp.bfloat16


def _leaky(h):
    return jnp.maximum(h, 0.01 * h)


def _row_mask():
    r = jax.lax.broadcasted_iota(jnp.int32, (NR, 1), 0)
    n = r // NP
    m = r - n * NP
    return ((n < NV) & (m < NV)).astype(F32)


def _p1_kernel(x_ref, w_ref, h_ref, s_ref, q_ref):
    x = x_ref[0]
    a = jnp.abs(x[:, None, :] - x[None, :, :]).reshape(NR, x.shape[1])
    a = a * _row_mask()
    h = jnp.dot(a.astype(BF16), w_ref[...], preferred_element_type=F32)
    h_ref[0] = h.astype(BF16)
    s_ref[0, 0, :] = jnp.sum(h, axis=0)
    q_ref[0, 0, :] = jnp.sum(h * h, axis=0)


def _mid_kernel(h_ref, sc_ref, sh_ref, w_ref, ho_ref, s_ref, q_ref):
    a = _leaky(h_ref[0].astype(F32) * sc_ref[0] + sh_ref[0]) * _row_mask()
    h = jnp.dot(a.astype(BF16), w_ref[...], preferred_element_type=F32)
    ho_ref[0] = h.astype(BF16)
    s_ref[0, 0, :] = jnp.sum(h, axis=0)
    q_ref[0, 0, :] = jnp.sum(h * h, axis=0)


def _p5a_kernel(h_ref, sc_ref, sh_ref, w_ref, b_ref, ho_ref):
    a = _leaky(h_ref[0].astype(F32) * sc_ref[0] + sh_ref[0])
    ho_ref[0] = jnp.dot(a.astype(BF16), w_ref[...],
                        preferred_element_type=F32) + b_ref[0]


def _p5b_kernel(x_ref, e_ref, wg1_ref, wg2_ref, bg_ref,
                gout_ref, gs_ref, gq_ref):
    x = x_ref[0]
    ii = jax.lax.broadcasted_iota(jnp.int32, (NP, NP), 0)
    jj = jax.lax.broadcasted_iota(jnp.int32, (NP, NP), 1)
    le = e_ref[0] + jnp.where(ii == jj, -1e8, 0.0) \
        + jnp.where(jj >= NV, -1e9, 0.0)
    mx = jnp.max(le, axis=1, keepdims=True)
    ex = jnp.exp(le - mx)
    w = ex / jnp.sum(ex, axis=1, keepdims=True)
    y = jnp.dot(w, x, preferred_element_type=F32)
    gout = (jnp.dot(x, wg1_ref[...], preferred_element_type=F32)
            + jnp.dot(y, wg2_ref[...], preferred_element_type=F32)
            + bg_ref[0])
    ni = jax.lax.broadcasted_iota(jnp.int32, (NP, 1), 0)
    gout = gout * (ni < NV).astype(F32)
    gout_ref[0] = gout
    gs_ref[0, 0, :] = jnp.sum(gout, axis=0)
    gq_ref[0, 0, :] = jnp.sum(gout * gout, axis=0)


def _p5last_kernel(x_ref, h_ref, sc_ref, sh_ref, w5_ref, b5_ref,
                   wg1_ref, wg2_ref, bg_ref, logit_ref, sig_ref):
    x = x_ref[0]
    a = _leaky(h_ref[0].astype(F32) * sc_ref[0] + sh_ref[0])
    e = jnp.dot(a.astype(BF16), w5_ref[...],
                preferred_element_type=F32) + b5_ref[0]
    ri = jax.lax.broadcasted_iota(jnp.int32, (NP, 1), 0)
    le = e + jnp.where(ri == 0, -1e8, 0.0) + jnp.where(ri >= NV, -1e9, 0.0)
    mx = jnp.max(le, axis=0, keepdims=True)
    ex = jnp.exp(le - mx)
    w = ex / jnp.sum(ex, axis=0, keepdims=True)
    y = jax.lax.dot_general(w, x, (((0,), (0,)), ((), ())),
                            preferred_element_type=F32)
    gl = (jnp.dot(x[0:1, :], wg1_ref[...], preferred_element_type=F32)
          + jnp.dot(y, wg2_ref[...], preferred_element_type=F32)
          + bg_ref[0])
    logit_ref[0, 0, :] = gl[0]
    sig_ref[0, 0, :] = (1.0 / (1.0 + jnp.exp(-gl)))[0]


def _bn_act_kernel(g_ref, sc_ref, sh_ref, out_ref):
    ni = jax.lax.broadcasted_iota(jnp.int32, (NP, 1), 0)
    vmask = (ni < NV).astype(F32)
    out_ref[0] = _leaky(g_ref[0] * sc_ref[0] + sh_ref[0]) * vmask


def _bspec(shape):
    nd = len(shape)
    return pl.BlockSpec(shape, lambda b: (0,) * nd)


def _row_specs(shape):
    return pl.BlockSpec((1,) + shape[1:], lambda b: (b,) + (0,) * (len(shape) - 1))


def _finalize(s, q, b, g, beta, cnt):
    mean = (s + cnt * b) / cnt
    ex2 = (q + 2.0 * b * s + cnt * b * b) / cnt
    var = ex2 - mean * mean
    sc = g * jax.lax.rsqrt(var + 1e-5)
    sh = (b - mean) * sc + beta
    return sc.reshape(1, -1), sh.reshape(1, -1)


def _wcompute_gconv(xp, wc, gcp, last):
    B, _, F = xp.shape
    ws = [w.astype(BF16) for w in wc["w"]]
    bs = wc["b"]
    cnt = float(B * NV * NV)
    dims = [w.shape[1] for w in wc["w"]]

    def rspec(shape):
        return pl.BlockSpec((1,) + tuple(shape[1:]),
                            lambda b: (b,) + (0,) * (len(shape) - 1))

    h_shape = (B, NR, dims[0])
    s_shape = (B, 1, dims[0])
    h1, s, q = pl.pallas_call(
        _p1_kernel, grid=(B,),
        in_specs=[rspec(xp.shape), _bspec(ws[0].shape)],
        out_specs=[rspec(h_shape), rspec(s_shape), rspec(s_shape)],
        out_shape=[jax.ShapeDtypeStruct(h_shape, BF16),
                   jax.ShapeDtypeStruct(s_shape, F32),
                   jax.ShapeDtypeStruct(s_shape, F32)])(xp, ws[0])
    sc, sh = _finalize(jnp.sum(s[:, 0, :], 0), jnp.sum(q[:, 0, :], 0),
                       bs[0], wc["g"][0], wc["beta"][0], cnt)

    h_prev = h1
    for k in range(1, 4):
        ho_shape = (B, NR, dims[k])
        so_shape = (B, 1, dims[k])
        h_next, s, q = pl.pallas_call(
            _mid_kernel, grid=(B,),
            in_specs=[rspec(h_prev.shape), _bspec(sc.shape),
                      _bspec(sh.shape), _bspec(ws[k].shape)],
            out_specs=[rspec(ho_shape), rspec(so_shape), rspec(so_shape)],
            out_shape=[jax.ShapeDtypeStruct(ho_shape, BF16),
                       jax.ShapeDtypeStruct(so_shape, F32),
                       jax.ShapeDtypeStruct(so_shape, F32)])(
                h_prev, sc, sh, ws[k])
        sc, sh = _finalize(jnp.sum(s[:, 0, :], 0), jnp.sum(q[:, 0, :], 0),
                           bs[k], wc["g"][k], wc["beta"][k], cnt)
        h_prev = h_next

    b5 = bs[4].reshape(1, -1)
    wg1, wg2 = gcp["w"][:F], gcp["w"][F:]
    bg = gcp["b"].reshape(1, -1)
    Fo = wg1.shape[1]

    if last:
        o_shape = (B, 1, Fo)
        logits, sig = pl.pallas_call(
            _p5last_kernel, grid=(B,),
            in_specs=[rspec(xp.shape),
                      pl.BlockSpec((1, NP, dims[3]), lambda b: (b, 0, 0)),
                      _bspec(sc.shape), _bspec(sh.shape),
                      _bspec(ws[4].shape), _bspec(b5.shape),
                      _bspec(wg1.shape), _bspec(wg2.shape), _bspec(bg.shape)],
            out_specs=[rspec(o_shape), rspec(o_shape)],
            out_shape=[jax.ShapeDtypeStruct(o_shape, F32)] * 2)(
                xp, h_prev, sc, sh, ws[4], b5, wg1, wg2, bg)
        return logits[:, 0, :], sig[:, 0, :]

    e_shape = (B, NR, 1)
    e_col = pl.pallas_call(
        _p5a_kernel, grid=(B,),
        in_specs=[rspec(h_prev.shape), _bspec(sc.shape), _bspec(sh.shape),
                  _bspec(ws[4].shape), _bspec(b5.shape)],
        out_specs=rspec(e_shape),
        out_shape=jax.ShapeDtypeStruct(e_shape, F32))(
            h_prev, sc, sh, ws[4], b5)
    e_grid = e_col.reshape(B, NP, NP)

    gout_shape = (B, NP, Fo)
    gs_shape = (B, 1, Fo)
    gout, gs, gq = pl.pallas_call(
        _p5b_kernel, grid=(B,),
        in_specs=[rspec(xp.shape), rspec(e_grid.shape),
                  _bspec(wg1.shape), _bspec(wg2.shape), _bspec(bg.shape)],
        out_specs=[rspec(gout_shape), rspec(gs_shape), rspec(gs_shape)],
        out_shape=[jax.ShapeDtypeStruct(gout_shape, F32),
                   jax.ShapeDtypeStruct(gs_shape, F32),
                   jax.ShapeDtypeStruct(gs_shape, F32)])(
            xp, e_grid, wg1, wg2, bg)
    gsc, gsh = _finalize(jnp.sum(gs[:, 0, :], 0), jnp.sum(gq[:, 0, :], 0),
                         jnp.zeros((Fo,), F32), gcp["g"], gcp["beta"],
                         float(B * NV))
    act = pl.pallas_call(
        _bn_act_kernel, grid=(B,),
        in_specs=[rspec(gout_shape), _bspec(gsc.shape), _bspec(gsh.shape)],
        out_specs=rspec(gout_shape),
        out_shape=jax.ShapeDtypeStruct(gout_shape, F32))(gout, gsc, gsh)
    return act


def kernel(z, zi_s, labels_yi, params):
    B = z.shape[0]
    zero_pad = jnp.zeros((1, B, labels_yi.shape[2]), dtype=labels_yi.dtype)
    lab_all = jnp.concatenate([zero_pad, labels_yi], axis=0)
    z_all = jnp.concatenate([z[None], zi_s], axis=0)
    nodes = jnp.transpose(jnp.concatenate([z_all, lab_all], axis=2), (1, 0, 2))
    xp = jnp.pad(nodes, ((0, 0), (0, NP - NV), (0, 0)))
    for i in range(2):
        act = _wcompute_gconv(xp, params["wc"][i], params["gc"][i], last=False)
        xp = jnp.concatenate([xp, act], axis=2)
    logits, sig = _wcompute_gconv(xp, params["wc"][2], params["gc"][2],
                                  last=True)
    return (sig, logits)
